# baseline (speedup 1.0000x reference)
import functools

import jax
import jax.numpy as jnp
from jax import lax
from jax.experimental import pallas as pl
from jax.experimental.pallas import tpu as pltpu
from jax.experimental.pallas import tpu_sc as plsc

_D = 64
_DP = 128
_RPD = 128
_K = 4
_GROUP = _K * _RPD
_TC = 128
_NTC = 4
_W1 = _NTC * _TC
_L = 16


def _wid():
    return lax.axis_index("s") * 2 + lax.axis_index("c")


def _cols_pad(v_all):
    return ((v_all + _TC - 1) // _TC + 7) // 8 * 8


@functools.lru_cache(maxsize=None)
def _build_detile(v_all, nw):
    full = v_all // _W1
    tail = v_all - full * _W1
    slots = 2 * (((full + nw - 1) // nw + 1) // 2)

    mesh = plsc.VectorSubcoreMesh(core_axis_name="c", subcore_axis_name="s")

    @functools.partial(
        pl.kernel,
        mesh=mesh,
        compiler_params=pltpu.CompilerParams(use_tc_tiling_on_sc=True),
        out_type=jax.ShapeDtypeStruct((_D, _cols_pad(v_all), _TC),
                                      jnp.float32),
        scratch_types=[
            pltpu.VMEM((2, _D, _NTC, _TC), jnp.float32),
            pltpu.SemaphoreType.DMA,
            pltpu.SemaphoreType.DMA,
            pltpu.SemaphoreType.DMA,
            pltpu.SemaphoreType.DMA,
        ],
    )
    def detile_kernel(tab_hbm, out_hbm, stage, si0, si1, so0, so1):
        wid = _wid()
        sis = (si0, si1)
        sos = (so0, so1)

        def chunk_of(t):
            return wid + t * nw

        def fire_in(t, b):
            c = chunk_of(t)

            @pl.when(c < full)
            def _():
                off = pl.multiple_of(c * _W1, _W1)
                for tc in range(_NTC):
                    pltpu.async_copy(
                        tab_hbm.at[:, pl.ds(off + tc * _TC, _TC)],
                        stage.at[b, :, tc], sis[b])

        def drain_in(b):
            pltpu.make_async_copy(
                out_hbm.at[:, pl.ds(0, _NTC), :], stage.at[b], sis[b]).wait()

        def fire_out(t, b):
            c = chunk_of(t)

            @pl.when(c < full)
            def _():
                off = pl.multiple_of(c * _NTC, _NTC)
                pltpu.async_copy(
                    stage.at[b], out_hbm.at[:, pl.ds(off, _NTC), :], sos[b])

        def drain_out(b):
            pltpu.make_async_copy(
                stage.at[b], out_hbm.at[:, pl.ds(0, _NTC), :], sos[b]).wait()

        fire_in(0, 0)

        def body(i, carry):
            for b in range(2):
                t = i * 2 + b

                @pl.when((t >= 1) & (chunk_of(t - 1) < full))
                def _():
                    drain_out(1 - b)

                fire_in(t + 1, 1 - b)

                @pl.when(chunk_of(t) < full)
                def _():
                    drain_in(b)
                    fire_out(t, b)

            return carry

        lax.fori_loop(0, slots // 2, body, 0)

        @pl.when(chunk_of(slots - 1) < full)
        def _():
            drain_out((slots - 1) % 2)

    return detile_kernel


@functools.lru_cache(maxsize=None)
def _build_transpose(v_all, nw):
    full = v_all // _TC
    tail = v_all - full * _TC
    slots = 2 * (((full + nw - 1) // nw + 1) // 2)
    celems = _TC * _D

    mesh = plsc.VectorSubcoreMesh(core_axis_name="c", subcore_axis_name="s")

    @functools.partial(
        pl.kernel,
        mesh=mesh,
        compiler_params=pltpu.CompilerParams(use_tc_tiling_on_sc=False,
                                             needs_layout_passes=False),
        out_type=jax.ShapeDtypeStruct((v_all * _D,), jnp.float32),
        scratch_types=[
            pltpu.VMEM((2, _D, _TC), jnp.float32),
            pltpu.VMEM((2, celems), jnp.float32),
            pltpu.VMEM((max(tail, 1), _D), jnp.float32),
            pltpu.SemaphoreType.DMA,
            pltpu.SemaphoreType.DMA,
            pltpu.SemaphoreType.DMA,
            pltpu.SemaphoreType.DMA,
        ],
    )
    def transpose_kernel(kmaj_hbm, tail_hbm, out_hbm, stage, obuf, tailbuf,
                         si0, si1, so0, so1):
        wid = _wid()
        sis = (si0, si1)
        sos = (so0, so1)
        cvec = lax.iota(jnp.int32, _L) * _D

        def chunk_of(t):
            return wid + t * nw

        def fire_in(t, b):
            c = chunk_of(t)

            @pl.when(c < full)
            def _():
                pltpu.async_copy(
                    kmaj_hbm.at[:, c, pl.ds(0, _TC)], stage.at[b], sis[b])

        def drain_in(b):
            pltpu.make_async_copy(
                kmaj_hbm.at[:, 0, pl.ds(0, _TC)], stage.at[b], sis[b]).wait()

        def transpose(b, nq):
            for k in range(_D):
                for q in range(nq):
                    vals = stage[b, k, pl.ds(q * _L, _L)]
                    idx = cvec + (q * _L * _D + k)
                    plsc.store_scatter(obuf.at[b], [idx], vals)

        def fire_out(t, b):
            c = chunk_of(t)

            @pl.when(c < full)
            def _():
                off = pl.multiple_of(c * celems, celems)
                pltpu.async_copy(
                    obuf.at[b], out_hbm.at[pl.ds(off, celems)], sos[b])

        def drain_out(b):
            pltpu.make_async_copy(
                obuf.at[b], out_hbm.at[pl.ds(0, celems)], sos[b]).wait()

        fire_in(0, 0)

        def body(i, carry):
            for b in range(2):
                t = i * 2 + b
                fire_in(t + 1, 1 - b)

                @pl.when((t >= 2) & (chunk_of(t - 2) < full))
                def _():
                    drain_out(b)

                @pl.when(chunk_of(t) < full)
                def _():
                    drain_in(b)
                    transpose(b, _TC // _L)
                    fire_out(t, b)

            return carry

        lax.fori_loop(0, slots // 2, body, 0)

        for t in (slots - 2, slots - 1):
            @pl.when(chunk_of(t) < full)
            def _():
                drain_out(t % 2)

        if tail:
            @pl.when(wid == nw - 1)
            def _():
                pltpu.sync_copy(tail_hbm, tailbuf)
                for vv in range(tail):
                    pltpu.async_copy(
                        tailbuf.at[vv],
                        out_hbm.at[pl.ds((full * _TC + vv) * _D, _D)], si0)
                for vv in range(tail):
                    pltpu.make_async_copy(
                        out_hbm.at[pl.ds(0, _D)], tailbuf.at[vv], si0).wait()

    return transpose_kernel


@functools.lru_cache(maxsize=None)
def _build_gather(n_rows, n_cols, nw):
    rows_per_w = n_rows // nw

    mesh = plsc.VectorSubcoreMesh(core_axis_name="c", subcore_axis_name="s")

    @functools.partial(
        pl.kernel,
        mesh=mesh,
        compiler_params=pltpu.CompilerParams(use_tc_tiling_on_sc=False),
        out_type=jax.ShapeDtypeStruct((n_rows, (n_cols + 7) // 8, 8, _DP),
                                      jnp.float32),
        scratch_types=[
            pltpu.VMEM((n_cols, rows_per_w), jnp.int32),
            pltpu.VMEM((2, _GROUP, _D), jnp.float32),
            pltpu.SemaphoreType.DMA,
            pltpu.SemaphoreType.DMA,
        ],
    )
    def gather_kernel(idx_hbm, table_hbm, out_hbm, idx_v, rows_v, sem0, sem1):
        wid = _wid()
        base = wid * rows_per_w
        pltpu.sync_copy(idx_hbm.at[:, pl.ds(base, rows_per_w)], idx_v)

        sems = (sem0, sem1)

        def fire(g, b):
            for jj in range(_K):
                pltpu.async_copy(
                    table_hbm.at[idx_v.at[g, pl.ds(jj * _RPD, _RPD)]],
                    rows_v.at[b, pl.ds(jj * _RPD, _RPD)],
                    sems[b],
                )

        def drain(b):
            pltpu.make_async_copy(
                table_hbm.at[pl.ds(0, _GROUP)],
                rows_v.at[b],
                sems[b],
            ).wait()

        def store(g, b):
            pltpu.sync_copy(
                rows_v.at[b],
                out_hbm.at[pl.ds(base, _GROUP), g // 8, g % 8, pl.ds(0, _D)],
            )

        fire(0, 0)

        def body(i, carry):
            for b in range(2):
                g = i * 2 + b

                @pl.when(g + 1 < n_cols)
                def _():
                    fire(g + 1, 1 - b)

                drain(b)
                store(g, b)
            return carry

        lax.fori_loop(0, n_cols // 2, body, 0)

    return gather_kernel


def kernel(node_ids, embedding_weight):
    r, c = node_ids.shape
    v, d = embedding_weight.shape
    info = plsc.get_sparse_core_info()
    nw = info.num_cores * info.num_subcores
    ids_t = node_ids.astype(jnp.int32).T
    tab_t = embedding_weight.T
    n_tail = v % _TC
    tail_rows = embedding_weight[v - max(n_tail, 1):, :]
    kmaj = _build_detile(v, nw)(tab_t)
    flat = _build_transpose(v, nw)(kmaj, tail_rows)
    table = flat.reshape(v, d)
    out4 = _build_gather(r, c, nw)(ids_t, table)
    return out4.reshape(r, 8 * ((c + 7) // 8), _DP)[:, :c, :d]


# speedup vs baseline: 1.1322x; 1.1322x over previous
import functools

import jax
import jax.numpy as jnp
from jax import lax
from jax.experimental import pallas as pl
from jax.experimental.pallas import tpu as pltpu
from jax.experimental.pallas import tpu_sc as plsc

_D = 64
_DP = 128
_RPD = 128
_K = 4
_GROUP = _K * _RPD
_TC = 128
_L = 16


def _wid():
    return lax.axis_index("s") * 2 + lax.axis_index("c")


@functools.lru_cache(maxsize=None)
def _build_tabprep(v_all, nw):
    full = v_all // _TC
    tail = v_all - full * _TC
    slots = 2 * (((full + nw - 1) // nw + 1) // 2)
    celems = _TC * _D

    mesh = plsc.VectorSubcoreMesh(core_axis_name="c", subcore_axis_name="s")

    @functools.partial(
        pl.kernel,
        mesh=mesh,
        compiler_params=pltpu.CompilerParams(use_tc_tiling_on_sc=True,
                                             needs_layout_passes=False),
        out_type=jax.ShapeDtypeStruct((v_all * _D,), jnp.float32),
        scratch_types=[
            pltpu.VMEM((2, _D, _TC), jnp.float32),
            pltpu.VMEM((celems,), jnp.float32),
            pltpu.VMEM((celems,), jnp.float32),
            pltpu.VMEM((max(tail, 1), _DP), jnp.float32),
            pltpu.SemaphoreType.DMA,
            pltpu.SemaphoreType.DMA,
            pltpu.SemaphoreType.DMA,
            pltpu.SemaphoreType.DMA,
        ],
    )
    def tabprep_kernel(tab_hbm, tail_hbm, out_hbm, stage, obuf0, obuf1,
                       tailbuf, si0, si1, so0, so1):
        wid = _wid()
        sis = (si0, si1)
        sos = (so0, so1)
        obufs = (obuf0, obuf1)
        cvec = lax.iota(jnp.int32, _L) * _D

        def chunk_of(t):
            return wid + t * nw

        def fire_in(t, b):
            c = chunk_of(t)

            @pl.when(c < full)
            def _():
                off = pl.multiple_of(c * _TC, _TC)
                pltpu.async_copy(
                    tab_hbm.at[:, pl.ds(off, _TC)], stage.at[b], sis[b])

        def drain_in(b):
            pltpu.make_async_copy(
                tab_hbm.at[:, pl.ds(0, _TC)], stage.at[b], sis[b]).wait()

        def transpose(b, nq):
            span = (_L - 1) * _D + 8
            cvecs = [cvec + r for r in range(8)]

            def loads(k):
                return [stage[b, k, pl.ds(q * _L, _L)] for q in range(nq)]

            def stores(k, vals):
                for q in range(nq):
                    off = q * _L * _D + (k // 8) * 8
                    plsc.store_scatter(
                        obufs[b].at[pl.ds(off, span)], [cvecs[k % 8]],
                        vals[q])

            vals = loads(0)
            for k in range(1, _D):
                nxt = loads(k)
                stores(k - 1, vals)
                vals = nxt
            stores(_D - 1, vals)

        def fire_out(t, b):
            c = chunk_of(t)

            @pl.when(c < full)
            def _():
                off = pl.multiple_of(c * celems, celems)
                pltpu.async_copy(
                    obufs[b], out_hbm.at[pl.ds(off, celems)], sos[b])

        def drain_out(b):
            pltpu.make_async_copy(
                obufs[b], out_hbm.at[pl.ds(0, celems)], sos[b]).wait()

        fire_in(0, 0)

        def body(i, carry):
            for b in range(2):
                t = i * 2 + b
                fire_in(t + 1, 1 - b)

                @pl.when((t >= 2) & (chunk_of(t - 2) < full))
                def _():
                    drain_out(b)

                @pl.when(chunk_of(t) < full)
                def _():
                    drain_in(b)
                    transpose(b, _TC // _L)
                    fire_out(t, b)

            return carry

        lax.fori_loop(0, slots // 2, body, 0)

        for t in (slots - 2, slots - 1):
            @pl.when(chunk_of(t) < full)
            def _():
                drain_out(t % 2)

        if tail:
            @pl.when(wid == nw - 1)
            def _():
                pltpu.sync_copy(tail_hbm, tailbuf)
                for vv in range(tail):
                    pltpu.async_copy(
                        tailbuf.at[vv, pl.ds(0, _D)],
                        out_hbm.at[pl.ds((full * _TC + vv) * _D, _D)], si0)
                for vv in range(tail):
                    pltpu.make_async_copy(
                        out_hbm.at[pl.ds(0, _D)],
                        tailbuf.at[vv, pl.ds(0, _D)], si0).wait()

    return tabprep_kernel


@functools.lru_cache(maxsize=None)
def _build_gather(n_rows, n_cols, nw):
    rows_per_w = n_rows // nw

    mesh = plsc.VectorSubcoreMesh(core_axis_name="c", subcore_axis_name="s")

    @functools.partial(
        pl.kernel,
        mesh=mesh,
        compiler_params=pltpu.CompilerParams(use_tc_tiling_on_sc=False),
        out_type=jax.ShapeDtypeStruct((n_rows, (n_cols + 7) // 8, 8, _DP),
                                      jnp.float32),
        scratch_types=[
            pltpu.VMEM((n_cols, rows_per_w), jnp.int32),
            pltpu.VMEM((2, _GROUP, _D), jnp.float32),
            pltpu.SemaphoreType.DMA,
            pltpu.SemaphoreType.DMA,
        ],
    )
    def gather_kernel(idx_hbm, table_hbm, out_hbm, idx_v, rows_v, sem0, sem1):
        wid = _wid()
        base = wid * rows_per_w
        pltpu.sync_copy(idx_hbm.at[:, pl.ds(base, rows_per_w)], idx_v)

        sems = (sem0, sem1)

        def fire(g, b):
            for jj in range(_K):
                pltpu.async_copy(
                    table_hbm.at[idx_v.at[g, pl.ds(jj * _RPD, _RPD)]],
                    rows_v.at[b, pl.ds(jj * _RPD, _RPD)],
                    sems[b],
                )

        def drain(b):
            pltpu.make_async_copy(
                table_hbm.at[pl.ds(0, _GROUP)],
                rows_v.at[b],
                sems[b],
            ).wait()

        def store(g, b):
            pltpu.sync_copy(
                rows_v.at[b],
                out_hbm.at[pl.ds(base, _GROUP), g // 8, g % 8, pl.ds(0, _D)],
            )

        fire(0, 0)

        def body(i, carry):
            for b in range(2):
                g = i * 2 + b

                @pl.when(g + 1 < n_cols)
                def _():
                    fire(g + 1, 1 - b)

                drain(b)
                store(g, b)
            return carry

        lax.fori_loop(0, n_cols // 2, body, 0)

    return gather_kernel


def kernel(node_ids, embedding_weight):
    r, c = node_ids.shape
    v, d = embedding_weight.shape
    info = plsc.get_sparse_core_info()
    nw = info.num_cores * info.num_subcores
    ids_t = node_ids.astype(jnp.int32).T
    tab_t = embedding_weight.T
    n_tail = v % _TC
    tail_rows = embedding_weight[v - max(n_tail, 1):, :]
    tail128 = jnp.pad(tail_rows, ((0, 0), (0, _DP - d)))
    flat = _build_tabprep(v, nw)(tab_t, tail128)
    table = flat.reshape(v, d)
    out4 = _build_gather(r, c, nw)(ids_t, table)
    return out4.reshape(r, 8 * ((c + 7) // 8), _DP)[:, :c, :d]


# speedup vs baseline: 1.1364x; 1.0037x over previous
import functools

import jax
import jax.numpy as jnp
from jax import lax
from jax.experimental import pallas as pl
from jax.experimental.pallas import tpu as pltpu
from jax.experimental.pallas import tpu_sc as plsc

_D = 64
_DP = 128
_RPD = 128
_K = 4
_GROUP = _K * _RPD
_TC = 128
_L = 16


def _wid():
    return lax.axis_index("s") * 2 + lax.axis_index("c")


@functools.lru_cache(maxsize=None)
def _build_tabprep(v_all, nw):
    full = v_all // _TC
    tail = v_all - full * _TC
    slots = 4 * (((full + nw - 1) // nw + 3) // 4)
    celems = _TC * _D

    mesh = plsc.VectorSubcoreMesh(core_axis_name="c", subcore_axis_name="s")

    @functools.partial(
        pl.kernel,
        mesh=mesh,
        compiler_params=pltpu.CompilerParams(use_tc_tiling_on_sc=True,
                                             needs_layout_passes=False),
        out_type=jax.ShapeDtypeStruct((v_all * _D,), jnp.float32),
        scratch_types=[
            pltpu.VMEM((4, _D, _TC), jnp.float32),
            pltpu.VMEM((celems,), jnp.float32),
            pltpu.VMEM((celems,), jnp.float32),
            pltpu.VMEM((celems,), jnp.float32),
            pltpu.VMEM((celems,), jnp.float32),
            pltpu.VMEM((max(tail, 1), _DP), jnp.float32),
            pltpu.SemaphoreType.DMA,
            pltpu.SemaphoreType.DMA,
            pltpu.SemaphoreType.DMA,
            pltpu.SemaphoreType.DMA,
            pltpu.SemaphoreType.DMA,
            pltpu.SemaphoreType.DMA,
            pltpu.SemaphoreType.DMA,
            pltpu.SemaphoreType.DMA,
        ],
    )
    def tabprep_kernel(tab_hbm, tail_hbm, out_hbm, stage, obuf0, obuf1,
                       obuf2, obuf3, tailbuf, si0, si1, si2, si3,
                       so0, so1, so2, so3):
        wid = _wid()
        sis = (si0, si1, si2, si3)
        sos = (so0, so1, so2, so3)
        obufs = (obuf0, obuf1, obuf2, obuf3)
        cvec = lax.iota(jnp.int32, _L) * _D

        def chunk_of(t):
            return wid + t * nw

        def fire_in(t, b):
            c = chunk_of(t)

            @pl.when(c < full)
            def _():
                off = pl.multiple_of(c * _TC, _TC)
                pltpu.async_copy(
                    tab_hbm.at[:, pl.ds(off, _TC)], stage.at[b], sis[b])

        def drain_in(b):
            pltpu.make_async_copy(
                tab_hbm.at[:, pl.ds(0, _TC)], stage.at[b], sis[b]).wait()

        def transpose(b, nq):
            span = (_L - 1) * _D + 8
            cvecs = [cvec + r for r in range(8)]

            def loads(k):
                return [stage[b, k, pl.ds(q * _L, _L)] for q in range(nq)]

            def stores(k, vals):
                for q in range(nq):
                    off = q * _L * _D + (k // 8) * 8
                    plsc.store_scatter(
                        obufs[b].at[pl.ds(off, span)], [cvecs[k % 8]],
                        vals[q])

            vals = loads(0)
            for k in range(1, _D):
                nxt = loads(k)
                stores(k - 1, vals)
                vals = nxt
            stores(_D - 1, vals)

        def fire_out(t, b):
            c = chunk_of(t)

            @pl.when(c < full)
            def _():
                off = pl.multiple_of(c * celems, celems)
                pltpu.async_copy(
                    obufs[b], out_hbm.at[pl.ds(off, celems)], sos[b])

        def drain_out(b):
            pltpu.make_async_copy(
                obufs[b], out_hbm.at[pl.ds(0, celems)], sos[b]).wait()

        for p in range(3):
            fire_in(p, p)

        def body(i, carry):
            for b in range(4):
                t = i * 4 + b
                fire_in(t + 3, (b + 3) % 4)

                @pl.when((t >= 4) & (chunk_of(t - 4) < full))
                def _():
                    drain_out(b)

                @pl.when(chunk_of(t) < full)
                def _():
                    drain_in(b)
                    transpose(b, _TC // _L)
                    fire_out(t, b)

            return carry

        lax.fori_loop(0, slots // 4, body, 0)

        for t in range(slots - 4, slots):
            @pl.when(chunk_of(t) < full)
            def _():
                drain_out(t % 4)

        if tail:
            @pl.when(wid == nw - 1)
            def _():
                pltpu.sync_copy(tail_hbm, tailbuf)
                for vv in range(tail):
                    pltpu.async_copy(
                        tailbuf.at[vv, pl.ds(0, _D)],
                        out_hbm.at[pl.ds((full * _TC + vv) * _D, _D)], si0)
                for vv in range(tail):
                    pltpu.make_async_copy(
                        out_hbm.at[pl.ds(0, _D)],
                        tailbuf.at[vv, pl.ds(0, _D)], si0).wait()

    return tabprep_kernel


@functools.lru_cache(maxsize=None)
def _build_gather(n_rows, n_cols, nw):
    rows_per_w = n_rows // nw

    mesh = plsc.VectorSubcoreMesh(core_axis_name="c", subcore_axis_name="s")

    @functools.partial(
        pl.kernel,
        mesh=mesh,
        compiler_params=pltpu.CompilerParams(use_tc_tiling_on_sc=False),
        out_type=jax.ShapeDtypeStruct((n_rows, (n_cols + 7) // 8, 8, _DP),
                                      jnp.float32),
        scratch_types=[
            pltpu.VMEM((n_cols, rows_per_w), jnp.int32),
            pltpu.VMEM((2, _GROUP, _D), jnp.float32),
            pltpu.SemaphoreType.DMA,
            pltpu.SemaphoreType.DMA,
        ],
    )
    def gather_kernel(idx_hbm, table_hbm, out_hbm, idx_v, rows_v, sem0, sem1):
        wid = _wid()
        base = wid * rows_per_w
        pltpu.sync_copy(idx_hbm.at[:, pl.ds(base, rows_per_w)], idx_v)

        sems = (sem0, sem1)

        def fire(g, b):
            for jj in range(_K):
                pltpu.async_copy(
                    table_hbm.at[idx_v.at[g, pl.ds(jj * _RPD, _RPD)]],
                    rows_v.at[b, pl.ds(jj * _RPD, _RPD)],
                    sems[b],
                )

        def drain(b):
            pltpu.make_async_copy(
                table_hbm.at[pl.ds(0, _GROUP)],
                rows_v.at[b],
                sems[b],
            ).wait()

        def store(g, b):
            pltpu.sync_copy(
                rows_v.at[b],
                out_hbm.at[pl.ds(base, _GROUP), g // 8, g % 8, pl.ds(0, _D)],
            )

        fire(0, 0)

        def body(i, carry):
            for b in range(2):
                g = i * 2 + b

                @pl.when(g + 1 < n_cols)
                def _():
                    fire(g + 1, 1 - b)

                drain(b)
                store(g, b)
            return carry

        lax.fori_loop(0, n_cols // 2, body, 0)

    return gather_kernel


def kernel(node_ids, embedding_weight):
    r, c = node_ids.shape
    v, d = embedding_weight.shape
    info = plsc.get_sparse_core_info()
    nw = info.num_cores * info.num_subcores
    ids_t = node_ids.astype(jnp.int32).T
    tab_t = embedding_weight.T
    n_tail = v % _TC
    tail_rows = embedding_weight[v - max(n_tail, 1):, :]
    tail128 = jnp.pad(tail_rows, ((0, 0), (0, _DP - d)))
    flat = _build_tabprep(v, nw)(tab_t, tail128)
    table = flat.reshape(v, d)
    out4 = _build_gather(r, c, nw)(ids_t, table)
    return out4.reshape(r, 8 * ((c + 7) // 8), _DP)[:, :c, :d]


# speedup vs baseline: 1.7181x; 1.5119x over previous
import functools

import jax
import jax.numpy as jnp
from jax import lax
from jax.experimental import pallas as pl
from jax.experimental.pallas import tpu as pltpu
from jax.experimental.pallas import tpu_sc as plsc

_D = 64
_DP = 128
_RPD = 128
_K = 2
_GROUP = _K * _RPD


@functools.lru_cache(maxsize=None)
def _build(n_rows, n_cols, nw):
    rows_per_w = n_rows // nw
    gpc = rows_per_w // _GROUP
    ng = n_cols * gpc

    mesh = plsc.VectorSubcoreMesh(core_axis_name="c", subcore_axis_name="s")

    @functools.partial(
        pl.kernel,
        mesh=mesh,
        compiler_params=pltpu.CompilerParams(use_tc_tiling_on_sc=False),
        out_type=jax.ShapeDtypeStruct((n_rows, (n_cols + 7) // 8, 8, _DP),
                                      jnp.float32),
        scratch_types=[
            pltpu.VMEM((n_cols, rows_per_w), jnp.int32),
            pltpu.VMEM((2, _GROUP, _DP), jnp.float32),
            pltpu.SemaphoreType.DMA,
            pltpu.SemaphoreType.DMA,
        ],
    )
    def gather_kernel(idx_hbm, table_hbm, out_hbm, idx_v, rows_v, sem0, sem1):
        wid = lax.axis_index("s") * 2 + lax.axis_index("c")
        base = wid * rows_per_w
        pltpu.sync_copy(idx_hbm.at[:, pl.ds(base, rows_per_w)], idx_v)

        sems = (sem0, sem1)

        def fire(g, b):
            j = g // gpc
            half = g % gpc
            for jj in range(_K):
                pltpu.async_copy(
                    table_hbm.at[
                        idx_v.at[j, pl.ds(half * _GROUP + jj * _RPD, _RPD)]],
                    rows_v.at[b, pl.ds(jj * _RPD, _RPD)],
                    sems[b],
                )

        def drain(b):
            pltpu.make_async_copy(
                table_hbm.at[pl.ds(0, _GROUP)],
                rows_v.at[b],
                sems[b],
            ).wait()

        def store(g, b):
            j = g // gpc
            half = g % gpc
            pltpu.sync_copy(
                rows_v.at[b],
                out_hbm.at[pl.ds(base + half * _GROUP, _GROUP), j // 8, j % 8],
            )

        fire(0, 0)

        def body(i, carry):
            for b in range(2):
                g = i * 2 + b

                @pl.when(g + 1 < ng)
                def _():
                    fire(g + 1, 1 - b)

                drain(b)
                store(g, b)
            return carry

        lax.fori_loop(0, ng // 2, body, 0)

    return gather_kernel


def kernel(node_ids, embedding_weight):
    r, c = node_ids.shape
    d = embedding_weight.shape[1]
    info = plsc.get_sparse_core_info()
    nw = info.num_cores * info.num_subcores
    ids_t = node_ids.astype(jnp.int32).T
    tab128 = jnp.pad(embedding_weight, ((0, 0), (0, _DP - d)))
    out4 = _build(r, c, nw)(ids_t, tab128)
    return out4.reshape(r, 8 * ((c + 7) // 8), _DP)[:, :c, :d]
